# baseline (speedup 1.0000x reference)
import functools

import jax
import jax.numpy as jnp
from jax import lax
from jax.experimental import pallas as pl
from jax.experimental.pallas import tpu as pltpu
from jax.experimental.pallas import tpu_sc as plsc

N = 10000
E = 320000
D = 128
NW = 32
N_PAD = 10240
CHUNK = 128
N_CHUNKS = 2560
C_TILE = 160
S = 32
E_PAD = N_CHUNKS * CHUNK
E_PER_TILE = E_PAD // NW
ROWS_PER_SUB = N_PAD // 16
R_BLK = 1024
GRID = N_PAD // R_BLK

_mesh = plsc.VectorSubcoreMesh(core_axis_name="c", subcore_axis_name="s")


@functools.partial(
    pl.kernel,
    out_type=jax.ShapeDtypeStruct((NW, N_PAD), jnp.float32),
    mesh=_mesh,
    compiler_params=pltpu.CompilerParams(needs_layout_passes=False),
    scratch_types=[
        pltpu.VMEM((E_PER_TILE,), jnp.int32),
        pltpu.VMEM((N_PAD,), jnp.float32),
    ],
)
def _deg_kernel(dst_hbm, degp_hbm, dst_v, deg_v):
    cid = lax.axis_index("c")
    sid = lax.axis_index("s")
    wid = sid * 2 + cid
    pltpu.sync_copy(dst_hbm.at[pl.ds(wid * E_PER_TILE, E_PER_TILE)], dst_v)
    zeros16 = jnp.zeros((16,), jnp.float32)
    ones16 = jnp.ones((16,), jnp.float32)

    def zbody(i, carry):
        deg_v[pl.ds(i * 16, 16)] = zeros16
        return carry

    lax.fori_loop(0, N_PAD // 16, zbody, 0)

    def abody(i, carry):
        idx = dst_v[pl.ds(i * 16, 16)]
        plsc.addupdate_scatter(deg_v, [idx], ones16)
        return carry

    lax.fori_loop(0, E_PER_TILE // 16, abody, 0)
    pltpu.sync_copy(deg_v, degp_hbm.at[wid])


@functools.partial(
    pl.kernel,
    out_type=jax.ShapeDtypeStruct((N_PAD, D), jnp.float32),
    mesh=_mesh,
    scratch_types=[
        pltpu.VMEM((S, CHUNK), jnp.int32),
        pltpu.VMEM((S, CHUNK), jnp.int32),
        pltpu.VMEM((CHUNK, D), jnp.float32),
        pltpu.VMEM((CHUNK, D), jnp.float32),
        pltpu.VMEM_SHARED((N_PAD, D), jnp.float32),
        pltpu.SemaphoreType.DMA,
        pltpu.SemaphoreType.DMA,
        pltpu.SemaphoreType.DMA,
        pltpu.SemaphoreType.DMA,
    ],
)
def _scatter_kernel(h_hbm, src_hbm, dst_hbm, acc_hbm, src_v, dst_v, buf, buf1,
                    acc_sh, g0, g1, s0, s1):
    cid = lax.axis_index("c")
    sid = lax.axis_index("s")

    @pl.when(cid == 0)
    def _run():
        zeros16 = jnp.zeros((16,), jnp.float32)

        def zbody(i, carry):
            for j in range(D // 16):
                buf[i, pl.ds(j * 16, 16)] = zeros16
            return carry

        lax.fori_loop(0, CHUNK, zbody, 0)
        for k in range(ROWS_PER_SUB // CHUNK):
            pltpu.sync_copy(
                buf, acc_sh.at[pl.ds(sid * ROWS_PER_SUB + k * CHUNK, CHUNK)])
        plsc.subcore_barrier()

        base_chunk = sid * C_TILE

        def stage_body(st, carry):
            chunk0 = base_chunk + st * S
            pltpu.sync_copy(src_hbm.at[pl.ds(chunk0, S)], src_v)
            pltpu.sync_copy(dst_hbm.at[pl.ds(chunk0, S)], dst_v)
            pltpu.async_copy(h_hbm.at[src_v.at[0]], buf, g0)

            def body(t, carry2):
                c0 = 2 * t
                c1 = 2 * t + 1
                pltpu.make_async_copy(h_hbm.at[src_v.at[c0]], buf, g0).wait()
                pltpu.async_copy(h_hbm.at[src_v.at[c1]], buf1, g1)
                pltpu.async_copy(buf, acc_sh.at[dst_v.at[c0]], s0, add=True)
                pltpu.make_async_copy(h_hbm.at[src_v.at[c1]], buf1, g1).wait()
                pltpu.make_async_copy(buf, acc_sh.at[dst_v.at[c0]], s0).wait()

                @pl.when(t < S // 2 - 1)
                def _():
                    pltpu.async_copy(h_hbm.at[src_v.at[c0 + 2]], buf, g0)

                pltpu.async_copy(buf1, acc_sh.at[dst_v.at[c1]], s1, add=True)
                pltpu.make_async_copy(buf1, acc_sh.at[dst_v.at[c1]], s1).wait()
                return carry2

            lax.fori_loop(0, S // 2, body, 0)
            return carry

        lax.fori_loop(0, C_TILE // S, stage_body, 0)
        plsc.subcore_barrier()
        pltpu.sync_copy(
            acc_sh.at[pl.ds(sid * ROWS_PER_SUB, ROWS_PER_SUB)],
            acc_hbm.at[pl.ds(sid * ROWS_PER_SUB, ROWS_PER_SUB)],
        )


def _dinv_from_degp(degp_blk):
    deg = jnp.sum(degp_blk, axis=1, keepdims=True) + 1.0
    return lax.rsqrt(deg)


def _stage1_body(x_ref, w1_ref, degp_ref, hs_ref):
    dinv = _dinv_from_degp(degp_ref[...])
    h = jnp.dot(x_ref[...], w1_ref[...], preferred_element_type=jnp.float32)
    hs_ref[...] = h * dinv


def _stage2_body(acc_ref, hs_ref, degp_ref, b1_ref, w2_ref, out_ref):
    dinv = _dinv_from_degp(degp_ref[...])
    pre = dinv * (acc_ref[...] + hs_ref[...]) + b1_ref[...]
    z = jnp.maximum(pre, 0.0)
    h2 = jnp.dot(z, w2_ref[...], preferred_element_type=jnp.float32)
    out_ref[...] = h2 * dinv


def _stage3_body(acc_ref, hs_ref, degp_ref, b2_ref, out_ref):
    dinv = _dinv_from_degp(degp_ref[...])
    out_ref[...] = dinv * (acc_ref[...] + hs_ref[...]) + b2_ref[...]


_row_spec = pl.BlockSpec((R_BLK, D), lambda i: (i, 0))
_degp_spec = pl.BlockSpec((R_BLK, NW), lambda i: (i, 0))
_w_spec = pl.BlockSpec((D, D), lambda i: (0, 0))
_b_spec = pl.BlockSpec((1, D), lambda i: (0, 0))

_stage1 = pl.pallas_call(
    _stage1_body,
    grid=(GRID,),
    in_specs=[_row_spec, _w_spec, _degp_spec],
    out_specs=_row_spec,
    out_shape=jax.ShapeDtypeStruct((N_PAD, D), jnp.float32),
)

_stage2 = pl.pallas_call(
    _stage2_body,
    grid=(GRID,),
    in_specs=[_row_spec, _row_spec, _degp_spec, _b_spec, _w_spec],
    out_specs=_row_spec,
    out_shape=jax.ShapeDtypeStruct((N_PAD, D), jnp.float32),
)

_stage3 = pl.pallas_call(
    _stage3_body,
    grid=(GRID,),
    in_specs=[_row_spec, _row_spec, _degp_spec, _b_spec],
    out_specs=_row_spec,
    out_shape=jax.ShapeDtypeStruct((N_PAD, D), jnp.float32),
)


@jax.jit
def kernel(x, edge_index, W1, b1, W2, b2):
    pad = E_PAD - E
    src = jnp.concatenate([edge_index[0], jnp.full((pad,), N, jnp.int32)])
    dst = jnp.concatenate([edge_index[1], jnp.full((pad,), N, jnp.int32)])
    src_t = src.reshape(N_CHUNKS, CHUNK)
    dst_t = dst.reshape(N_CHUNKS, CHUNK)
    x_pad = jnp.zeros((N_PAD, D), jnp.float32).at[:N].set(x)
    b1r = b1.reshape(1, D)
    b2r = b2.reshape(1, D)

    degp = _deg_kernel(dst)
    degp_t = degp.T

    h1s = _stage1(x_pad, W1, degp_t)
    acc1 = _scatter_kernel(h1s, src_t, dst_t)
    h2s = _stage2(acc1, h1s, degp_t, b1r, W2)
    acc2 = _scatter_kernel(h2s, src_t, dst_t)
    out = _stage3(acc2, h2s, degp_t, b2r)
    return out[:N]


# speedup vs baseline: 1.4723x; 1.4723x over previous
import functools

import jax
import jax.numpy as jnp
from jax import lax
from jax.experimental import pallas as pl
from jax.experimental.pallas import tpu as pltpu
from jax.experimental.pallas import tpu_sc as plsc

N = 10000
E = 320000
D = 128
NW = 32
N_PAD = 10240
CHUNK = 128
N_CHUNKS = 2560
CA = 128
CB = 32
S = 32
E_PAD = N_CHUNKS * CHUNK
E_PER_TILE = E_PAD // NW
ROWS_PER_SUB = N_PAD // 16
R_BLK = 1024
GRID = N_PAD // R_BLK

_mesh = plsc.VectorSubcoreMesh(core_axis_name="c", subcore_axis_name="s")


@functools.partial(
    pl.kernel,
    out_type=jax.ShapeDtypeStruct((NW, N_PAD), jnp.float32),
    mesh=_mesh,
    compiler_params=pltpu.CompilerParams(needs_layout_passes=False),
    scratch_types=[
        pltpu.VMEM((E_PER_TILE,), jnp.int32),
        pltpu.VMEM((N_PAD,), jnp.float32),
    ],
)
def _deg_kernel(dst_hbm, degp_hbm, dst_v, deg_v):
    cid = lax.axis_index("c")
    sid = lax.axis_index("s")
    wid = sid * 2 + cid
    pltpu.sync_copy(dst_hbm.at[pl.ds(wid * E_PER_TILE, E_PER_TILE)], dst_v)
    zeros16 = jnp.zeros((16,), jnp.float32)
    ones16 = jnp.ones((16,), jnp.float32)

    def zbody(i, carry):
        deg_v[pl.ds(i * 16, 16)] = zeros16
        return carry

    lax.fori_loop(0, N_PAD // 16, zbody, 0)

    def abody(i, carry):
        idx = dst_v[pl.ds(i * 16, 16)]
        plsc.addupdate_scatter(deg_v, [idx], ones16)
        return carry

    lax.fori_loop(0, E_PER_TILE // 16, abody, 0)
    pltpu.sync_copy(deg_v, degp_hbm.at[wid])


@functools.partial(
    pl.kernel,
    out_type=jax.ShapeDtypeStruct((2, N_PAD, D), jnp.float32),
    mesh=_mesh,
    scratch_types=[
        pltpu.VMEM((S, CHUNK), jnp.int32),
        pltpu.VMEM((S, CHUNK), jnp.int32),
        pltpu.VMEM((CHUNK, D), jnp.float32),
        pltpu.VMEM((CHUNK, D), jnp.float32),
        pltpu.VMEM_SHARED((N_PAD, D), jnp.float32),
        pltpu.SemaphoreType.DMA,
        pltpu.SemaphoreType.DMA,
        pltpu.SemaphoreType.DMA,
        pltpu.SemaphoreType.DMA,
    ],
)
def _scatter_kernel(h_hbm, src_hbm, dst_hbm, acc_hbm, src_v, dst_v, buf, buf1,
                    acc_sh, g0, g1, s0, s1):
    cid = lax.axis_index("c")
    sid = lax.axis_index("s")

    zeros16 = jnp.zeros((16,), jnp.float32)

    def zbody(i, carry):
        for j in range(D // 16):
            buf[i, pl.ds(j * 16, 16)] = zeros16
        return carry

    lax.fori_loop(0, CHUNK, zbody, 0)
    for k in range(ROWS_PER_SUB // CHUNK):
        pltpu.async_copy(
            buf, acc_sh.at[pl.ds(sid * ROWS_PER_SUB + k * CHUNK, CHUNK)], s0)
    for k in range(ROWS_PER_SUB // CHUNK):
        pltpu.make_async_copy(
            buf, acc_sh.at[pl.ds(sid * ROWS_PER_SUB + k * CHUNK, CHUNK)],
            s0).wait()
    plsc.subcore_barrier()

    n_stages = jnp.where(cid == 0, CA // S, CB // S)
    base_chunk = jnp.where(cid == 0, sid * CA, 16 * CA + sid * CB)

    def stage_body(st, carry):
        chunk0 = base_chunk + st * S
        pltpu.sync_copy(src_hbm.at[pl.ds(chunk0, S)], src_v)
        pltpu.sync_copy(dst_hbm.at[pl.ds(chunk0, S)], dst_v)
        pltpu.async_copy(h_hbm.at[src_v.at[0]], buf, g0)

        def body(t, carry2):
            c0 = 2 * t
            c1 = 2 * t + 1
            pltpu.make_async_copy(h_hbm.at[src_v.at[c0]], buf, g0).wait()
            pltpu.async_copy(h_hbm.at[src_v.at[c1]], buf1, g1)
            pltpu.async_copy(buf, acc_sh.at[dst_v.at[c0]], s0, add=True)
            pltpu.make_async_copy(h_hbm.at[src_v.at[c1]], buf1, g1).wait()
            pltpu.make_async_copy(buf, acc_sh.at[dst_v.at[c0]], s0).wait()

            @pl.when(t < S // 2 - 1)
            def _():
                pltpu.async_copy(h_hbm.at[src_v.at[c0 + 2]], buf, g0)

            pltpu.async_copy(buf1, acc_sh.at[dst_v.at[c1]], s1, add=True)
            pltpu.make_async_copy(buf1, acc_sh.at[dst_v.at[c1]], s1).wait()
            return carry2

        lax.fori_loop(0, S // 2, body, 0)
        return carry

    lax.fori_loop(0, n_stages, stage_body, 0)
    plsc.subcore_barrier()
    pltpu.sync_copy(
        acc_sh.at[pl.ds(sid * ROWS_PER_SUB, ROWS_PER_SUB)],
        acc_hbm.at[cid, pl.ds(sid * ROWS_PER_SUB, ROWS_PER_SUB)],
    )


def _dinv_from_degp(degp_blk):
    deg = jnp.sum(degp_blk, axis=1, keepdims=True) + 1.0
    return lax.rsqrt(deg)


def _stage1_body(x_ref, w1_ref, degp_ref, hs_ref):
    dinv = _dinv_from_degp(degp_ref[...])
    h = jnp.dot(x_ref[...], w1_ref[...], preferred_element_type=jnp.float32)
    hs_ref[...] = h * dinv


def _stage2_body(acc_ref, hs_ref, degp_ref, b1_ref, w2_ref, out_ref):
    dinv = _dinv_from_degp(degp_ref[...])
    pre = dinv * (acc_ref[0] + acc_ref[1] + hs_ref[...]) + b1_ref[...]
    z = jnp.maximum(pre, 0.0)
    h2 = jnp.dot(z, w2_ref[...], preferred_element_type=jnp.float32)
    out_ref[...] = h2 * dinv


def _stage3_body(acc_ref, hs_ref, degp_ref, b2_ref, out_ref):
    dinv = _dinv_from_degp(degp_ref[...])
    out_ref[...] = dinv * (acc_ref[0] + acc_ref[1] + hs_ref[...]) + b2_ref[...]


_row_spec = pl.BlockSpec((R_BLK, D), lambda i: (i, 0))
_acc_spec = pl.BlockSpec((2, R_BLK, D), lambda i: (0, i, 0))
_degp_spec = pl.BlockSpec((R_BLK, NW), lambda i: (i, 0))
_w_spec = pl.BlockSpec((D, D), lambda i: (0, 0))
_b_spec = pl.BlockSpec((1, D), lambda i: (0, 0))

_stage1 = pl.pallas_call(
    _stage1_body,
    grid=(GRID,),
    in_specs=[_row_spec, _w_spec, _degp_spec],
    out_specs=_row_spec,
    out_shape=jax.ShapeDtypeStruct((N_PAD, D), jnp.float32),
)

_stage2 = pl.pallas_call(
    _stage2_body,
    grid=(GRID,),
    in_specs=[_acc_spec, _row_spec, _degp_spec, _b_spec, _w_spec],
    out_specs=_row_spec,
    out_shape=jax.ShapeDtypeStruct((N_PAD, D), jnp.float32),
)

_stage3 = pl.pallas_call(
    _stage3_body,
    grid=(GRID,),
    in_specs=[_acc_spec, _row_spec, _degp_spec, _b_spec],
    out_specs=_row_spec,
    out_shape=jax.ShapeDtypeStruct((N_PAD, D), jnp.float32),
)


@jax.jit
def kernel(x, edge_index, W1, b1, W2, b2):
    pad = E_PAD - E
    src = jnp.concatenate([edge_index[0], jnp.full((pad,), N, jnp.int32)])
    dst = jnp.concatenate([edge_index[1], jnp.full((pad,), N, jnp.int32)])
    src_t = src.reshape(N_CHUNKS, CHUNK)
    dst_t = dst.reshape(N_CHUNKS, CHUNK)
    x_pad = jnp.zeros((N_PAD, D), jnp.float32).at[:N].set(x)
    b1r = b1.reshape(1, D)
    b2r = b2.reshape(1, D)

    degp = _deg_kernel(dst)
    degp_t = degp.T

    h1s = _stage1(x_pad, W1, degp_t)
    acc1 = _scatter_kernel(h1s, src_t, dst_t)
    h2s = _stage2(acc1, h1s, degp_t, b1r, W2)
    acc2 = _scatter_kernel(h2s, src_t, dst_t)
    out = _stage3(acc2, h2s, degp_t, b2r)
    return out[:N]
